# baseline (speedup 1.0000x reference)
import functools

import numpy as np
import jax
import jax.numpy as jnp
from jax import lax
from jax.experimental import pallas as pl
from jax.experimental.pallas import tpu as pltpu
from jax.experimental.pallas import tpu_sc as plsc

_DIM = 64
_MAX_LEN = 256

NC = 2
NS = 16
NW = NC * NS

CB = 2
IDX_W = 100


def _sinusoidal_pe(max_len, dim):
    pos = np.arange(max_len, dtype=np.float32)[:, None]
    i = np.arange(0, dim, 2, dtype=np.float32)[None, :]
    angle = pos / np.power(10000.0, i / dim)
    pe = np.zeros((max_len, dim), dtype=np.float32)
    pe[:, 0::2] = np.sin(angle)
    pe[:, 1::2] = np.cos(angle)
    return pe


@functools.partial(jax.jit, static_argnums=(3, 4))
def _run(ids2, pe_rep, table, B, S):
    D = table.shape[1]
    C = CB * S
    rows_pw = B // NW
    nchunk = rows_pw // CB
    k_per_chunk = C // IDX_W

    mesh = plsc.VectorSubcoreMesh(core_axis_name="c", subcore_axis_name="s")

    @functools.partial(
        pl.kernel,
        mesh=mesh,
        out_type=jax.ShapeDtypeStruct((B * S, D), jnp.float32),
        compiler_params=pltpu.CompilerParams(use_tc_tiling_on_sc=False),
        scratch_types=[
            pltpu.VMEM((k_per_chunk, IDX_W), jnp.int32),
            pltpu.VMEM_SHARED((C, D), jnp.float32),
            pltpu.VMEM((C, D), jnp.float32),
            pltpu.SemaphoreType.DMA,
        ],
    )
    def body(ids_hbm, pe_hbm, table_hbm, out_hbm, idx_v, pe_sh, rows_v, gsem):
        sid = lax.axis_index("s")
        wid = sid * NC + lax.axis_index("c")

        @pl.when(sid == 0)
        def _():
            pltpu.sync_copy(pe_hbm, pe_sh)

        plsc.subcore_barrier()
        idx_row_base = wid * (rows_pw * S // IDX_W)
        out_base = wid * rows_pw * S

        def chunk(c, carry):
            row0 = idx_row_base + c * k_per_chunk
            pltpu.sync_copy(ids_hbm.at[pl.ds(row0, k_per_chunk)], idx_v)
            pltpu.sync_copy(pe_sh, rows_v)
            descs = []
            for k in range(k_per_chunk):
                descs.append(pltpu.async_copy(
                    table_hbm.at[idx_v.at[k]],
                    rows_v.at[pl.ds(k * IDX_W, IDX_W)],
                    gsem, add=True))
            for d in descs:
                d.wait()
            out0 = out_base + c * C
            pltpu.sync_copy(rows_v, out_hbm.at[pl.ds(out0, C)])
            return carry

        lax.fori_loop(0, nchunk, chunk, 0)

    return body(ids2, pe_rep, table)


def kernel(input, tok_table):
    B, S = input.shape
    V, D = tok_table.shape
    pe = _sinusoidal_pe(_MAX_LEN, D)[:S]
    pe_rep = jnp.asarray(np.tile(pe, (CB, 1)))
    ids2 = input.reshape(B * S // IDX_W, IDX_W).astype(jnp.int32)
    out = _run(ids2, pe_rep, tok_table, B, S)
    return out.reshape(B, S, D)


# speedup vs baseline: 1.1029x; 1.1029x over previous
import functools

import numpy as np
import jax
import jax.numpy as jnp
from jax import lax
from jax.experimental import pallas as pl
from jax.experimental.pallas import tpu as pltpu
from jax.experimental.pallas import tpu_sc as plsc

_DIM = 64
_MAX_LEN = 256

NC = 2
NS = 16
NW = NC * NS

CB = 4
IDX_W = 100


def _sinusoidal_pe(max_len, dim):
    pos = np.arange(max_len, dtype=np.float32)[:, None]
    i = np.arange(0, dim, 2, dtype=np.float32)[None, :]
    angle = pos / np.power(10000.0, i / dim)
    pe = np.zeros((max_len, dim), dtype=np.float32)
    pe[:, 0::2] = np.sin(angle)
    pe[:, 1::2] = np.cos(angle)
    return pe


@functools.partial(jax.jit, static_argnums=(3, 4))
def _run(ids2, pe_rep, table, B, S):
    D = table.shape[1]
    C = CB * S
    rows_pw = B // NW
    nchunk = rows_pw // CB
    kpc = C // IDX_W

    mesh = plsc.VectorSubcoreMesh(core_axis_name="c", subcore_axis_name="s")

    @functools.partial(
        pl.kernel,
        mesh=mesh,
        out_type=jax.ShapeDtypeStruct((B * S, D), jnp.float32),
        compiler_params=pltpu.CompilerParams(use_tc_tiling_on_sc=False),
        scratch_types=[
            pltpu.VMEM((2, kpc, IDX_W), jnp.int32),
            pltpu.VMEM_SHARED((C, D), jnp.float32),
            pltpu.VMEM((2, C, D), jnp.float32),
            pltpu.SemaphoreType.DMA,
            pltpu.SemaphoreType.DMA,
            pltpu.SemaphoreType.DMA,
        ],
    )
    def body(ids_hbm, pe_hbm, table_hbm, out_hbm,
             idx_v, pe_sh, rows_v, isem, gsem, ssem):
        sid = lax.axis_index("s")
        wid = sid * NC + lax.axis_index("c")

        @pl.when(sid == 0)
        def _():
            pltpu.sync_copy(pe_hbm, pe_sh)

        plsc.subcore_barrier()

        idx_row_base = wid * nchunk * kpc
        out_base = wid * nchunk * C

        def idx_src(c):
            return ids_hbm.at[pl.ds(idx_row_base + c * kpc, kpc)]

        def out_dst(c):
            return out_hbm.at[pl.ds(out_base + c * C, C)]

        def fire_gathers(c, b):
            for k in range(kpc):
                pltpu.async_copy(table_hbm.at[idx_v.at[b, k]],
                                 rows_v.at[b].at[pl.ds(k * IDX_W, IDX_W)],
                                 gsem, add=True)

        def wait_gathers(b):
            for k in range(kpc):
                pltpu.make_async_copy(table_hbm.at[idx_v.at[b, k]],
                                      rows_v.at[b].at[pl.ds(k * IDX_W, IDX_W)],
                                      gsem).wait()

        pltpu.sync_copy(idx_src(0), idx_v.at[0])
        pltpu.async_copy(idx_src(1), idx_v.at[1], isem)
        pltpu.sync_copy(pe_sh, rows_v.at[0])
        fire_gathers(0, 0)

        def half(g, b):
            c = 2 * g + b
            nb = b ^ 1

            @pl.when(c >= 1)
            def _():
                pltpu.make_async_copy(rows_v.at[nb], out_dst(c - 1),
                                      ssem).wait()

            @pl.when(c + 1 < nchunk)
            def _():
                pltpu.sync_copy(pe_sh, rows_v.at[nb])
                pltpu.make_async_copy(idx_src(c + 1), idx_v.at[nb],
                                      isem).wait()

            wait_gathers(b)
            pltpu.async_copy(rows_v.at[b], out_dst(c), ssem)

            @pl.when(c + 2 < nchunk)
            def _():
                pltpu.async_copy(idx_src(c + 2), idx_v.at[b], isem)

            @pl.when(c + 1 < nchunk)
            def _():
                fire_gathers(c + 1, nb)

        def outer(g, carry):
            half(g, 0)
            half(g, 1)
            return carry

        lax.fori_loop(0, nchunk // 2, outer, 0)
        pltpu.make_async_copy(rows_v.at[(nchunk - 1) % 2],
                              out_dst(nchunk - 1), ssem).wait()

    return body(ids2, pe_rep, table)


def kernel(input, tok_table):
    B, S = input.shape
    V, D = tok_table.shape
    pe = _sinusoidal_pe(_MAX_LEN, D)[:S]
    pe_rep = jnp.asarray(np.tile(pe, (CB, 1)))
    ids2 = input.reshape(B * S // IDX_W, IDX_W).astype(jnp.int32)
    out = _run(ids2, pe_rep, tok_table, B, S)
    return out.reshape(B, S, D)
